# Initial kernel scaffold; baseline (speedup 1.0000x reference)
import jax
import jax.numpy as jnp
from jax.experimental import pallas as pl


def kernel(hidden_states, gate_w, w_gate, w_up, w_down):
    raise NotImplementedError("write your pallas kernel here")


# speedup vs baseline: 4.4682x; 4.4682x over previous
import functools

import jax
import jax.numpy as jnp
from jax.experimental import pallas as pl
from jax.experimental.pallas import tpu as pltpu

_T, _D, _E, _F, _K = 2048, 1024, 8, 512, 2
_TB = 256


def _dense_body(x_ref, gw_ref, wg_ref, wu_ref, wd_ref, o_ref):
    x = x_ref[...]
    logits = jnp.dot(x, gw_ref[...], preferred_element_type=jnp.float32)
    p = jax.nn.softmax(logits, axis=-1)
    col = jax.lax.broadcasted_iota(jnp.int32, (_TB, _E), 1)
    rank = jnp.zeros((_TB, _E), jnp.int32)
    for j in range(_E):
        pj = p[:, j:j + 1]
        rank = rank + (pj > p).astype(jnp.int32) \
                    + ((pj == p) & (j < col)).astype(jnp.int32)
    sel = rank < _K
    wsel = jnp.where(sel, p, 0.0)
    wsel = wsel / jnp.sum(wsel, axis=-1, keepdims=True)
    acc = jnp.zeros((_TB, _D), jnp.float32)
    for e in range(_E):
        g = jnp.dot(x, wg_ref[e], preferred_element_type=jnp.float32)
        u = jnp.dot(x, wu_ref[e], preferred_element_type=jnp.float32)
        h = (g * jax.nn.sigmoid(g)) * u
        y = jnp.dot(h, wd_ref[e], preferred_element_type=jnp.float32)
        acc = acc + wsel[:, e:e + 1] * y
    o_ref[...] = acc


def _moe_dense(hidden_states, gate_w, w_gate, w_up, w_down, interpret=False):
    return pl.pallas_call(
        _dense_body,
        grid=(_T // _TB,),
        in_specs=[
            pl.BlockSpec((_TB, _D), lambda i: (i, 0)),
            pl.BlockSpec((_D, _E), lambda i: (0, 0)),
            pl.BlockSpec((_E, _D, _F), lambda i: (0, 0, 0)),
            pl.BlockSpec((_E, _D, _F), lambda i: (0, 0, 0)),
            pl.BlockSpec((_E, _F, _D), lambda i: (0, 0, 0)),
        ],
        out_specs=pl.BlockSpec((_TB, _D), lambda i: (i, 0)),
        out_shape=jax.ShapeDtypeStruct((_T, _D), jnp.float32),
        interpret=interpret,
    )(hidden_states, gate_w, w_gate, w_up, w_down)


def kernel(hidden_states, gate_w, w_gate, w_up, w_down):
    return _moe_dense(hidden_states, gate_w, w_gate, w_up, w_down)
